# baseline (speedup 1.0000x reference)
import functools

import jax
import jax.numpy as jnp
from jax import lax
from jax.experimental import pallas as pl
from jax.experimental.pallas import tpu as pltpu
from jax.experimental.pallas import tpu_sc as plsc

D = 1024
SEQ = 2048
NC = 2
NS = 16
NW = NC * NS
K = 32
TB = 2048
EPS = 1e-12


def _make_sc_gather(ntok):
    tpw = ntok // NW
    nch = tpw // K
    mesh = plsc.VectorSubcoreMesh(core_axis_name="c", subcore_axis_name="s")

    @functools.partial(
        pl.kernel,
        out_type=jax.ShapeDtypeStruct((ntok, D), jnp.float32),
        mesh=mesh,
        compiler_params=pltpu.CompilerParams(needs_layout_passes=False),
        scratch_types=[
            pltpu.VMEM((2, K), jnp.int32),
            pltpu.VMEM((2, K, D), jnp.float32),
            pltpu.SemaphoreType.DMA((2,)),
            pltpu.SemaphoreType.DMA((2,)),
        ],
    )
    def sc_gather(ids_hbm, wtab_hbm, out_hbm, idx, rows, semg, semo):
        wid = lax.axis_index("s") * NC + lax.axis_index("c")
        base = wid * tpw

        def issue(c, b):
            pltpu.sync_copy(ids_hbm.at[pl.ds(base + c * K, K)], idx.at[b])
            return pltpu.async_copy(wtab_hbm.at[idx.at[b]], rows.at[b],
                                    semg.at[b])

        gat = {0: issue(0, 0)}
        out = {}
        for c in range(nch):
            b = c & 1
            if c + 1 < nch:
                if c >= 1:
                    out[c - 1].wait()
                gat[c + 1] = issue(c + 1, 1 - b)
            gat.pop(c).wait()
            out[c] = pltpu.async_copy(
                rows.at[b], out_hbm.at[pl.ds(base + c * K, K)], semo.at[b])
        out[nch - 2].wait()
        out[nch - 1].wait()

    return sc_gather


def _tc_ln_body(wsum_ref, pos_ref, tt_ref, tid_ref, g_ref, b_ref, out_ref):
    tidf = tid_ref[...]
    t0 = tt_ref[0:1, :]
    dt = tt_ref[1:2, :] - t0
    y = wsum_ref[...] + pos_ref[...] + (t0 + tidf * dt)
    mean = jnp.mean(y, axis=-1, keepdims=True)
    var = jnp.mean(y * y, axis=-1, keepdims=True) - mean * mean
    inv = lax.rsqrt(var + EPS)
    out_ref[...] = (y - mean) * inv * g_ref[...] + b_ref[...]


def _make_tc_ln(ntok):
    spb = SEQ // TB
    nb = ntok // SEQ
    tok = lambda j, i: (i * spb + j, 0)
    return pl.pallas_call(
        _tc_ln_body,
        grid=(spb, nb),
        in_specs=[
            pl.BlockSpec((TB, D), tok),
            pl.BlockSpec((TB, D), lambda j, i: (j, 0)),
            pl.BlockSpec((2, D), lambda j, i: (0, 0)),
            pl.BlockSpec((TB, 1), tok),
            pl.BlockSpec((1, D), lambda j, i: (0, 0)),
            pl.BlockSpec((1, D), lambda j, i: (0, 0)),
        ],
        out_specs=pl.BlockSpec((TB, D), tok),
        out_shape=jax.ShapeDtypeStruct((ntok, D), jnp.float32),
        input_output_aliases={0: 0},
    )


def kernel(input_ids, token_type_ids, word_emb, pos_emb, tt_emb, gamma, beta):
    b, seq = input_ids.shape
    ntok = b * seq
    ids_flat = input_ids.reshape(ntok).astype(jnp.int32)
    ttf = token_type_ids.reshape(ntok, 1).astype(jnp.float32)
    wsum = _make_sc_gather(ntok)(ids_flat, word_emb)
    out = _make_tc_ln(ntok)(wsum, pos_emb, tt_emb, ttf,
                            gamma.reshape(1, D), beta.reshape(1, D))
    return out.reshape(b, seq, D)


# speedup vs baseline: 1.0059x; 1.0059x over previous
import functools

import jax
import jax.numpy as jnp
from jax import lax
from jax.experimental import pallas as pl
from jax.experimental.pallas import tpu as pltpu
from jax.experimental.pallas import tpu_sc as plsc

D = 1024
SEQ = 2048
NC = 2
NS = 16
NW = NC * NS
K = 32
TB = 2048
EPS = 1e-12


def _make_sc_gather(ntok):
    tpw = ntok // NW
    nch = tpw // K
    mesh = plsc.VectorSubcoreMesh(core_axis_name="c", subcore_axis_name="s")

    @functools.partial(
        pl.kernel,
        out_type=jax.ShapeDtypeStruct((ntok, D), jnp.float32),
        mesh=mesh,
        compiler_params=pltpu.CompilerParams(needs_layout_passes=False),
        scratch_types=[
            pltpu.VMEM((2, K), jnp.int32),
            pltpu.VMEM((2, K, D), jnp.float32),
            pltpu.SemaphoreType.DMA((2,)),
            pltpu.SemaphoreType.DMA((2,)),
        ],
    )
    def sc_gather(ids_hbm, wtab_hbm, out_hbm, idx, rows, semg, semo):
        wid = lax.axis_index("s") * NC + lax.axis_index("c")
        base = wid * tpw

        def issue(c, b):
            pltpu.sync_copy(ids_hbm.at[pl.ds(base + c * K, K)], idx.at[b])
            return pltpu.async_copy(wtab_hbm.at[idx.at[b]], rows.at[b],
                                    semg.at[b])

        gat = {0: issue(0, 0)}
        out = {}
        for c in range(nch):
            b = c & 1
            if c + 1 < nch:
                if c >= 1:
                    out[c - 1].wait()
                gat[c + 1] = issue(c + 1, 1 - b)
            gat.pop(c).wait()
            out[c] = pltpu.async_copy(
                rows.at[b], out_hbm.at[pl.ds(base + c * K, K)], semo.at[b])
        out[nch - 2].wait()
        out[nch - 1].wait()

    return sc_gather


def _tc_ln_body(wsum_ref, pos_ref, tt_ref, tid_ref, g_ref, b_ref, out_ref):
    tidf = tid_ref[...]
    t0 = tt_ref[0:1, :]
    dt = tt_ref[1:2, :] - t0
    y = wsum_ref[...] + pos_ref[...] + (t0 + tidf * dt)
    mean = jnp.mean(y, axis=-1, keepdims=True)
    var = jnp.mean(y * y, axis=-1, keepdims=True) - mean * mean
    inv = lax.rsqrt(var + EPS)
    out_ref[...] = (y - mean) * inv * g_ref[...] + b_ref[...]


def _make_tc_ln(ntok):
    spb = SEQ // TB
    nb = ntok // SEQ
    tok = lambda j, i: (i * spb + j, 0)
    return pl.pallas_call(
        _tc_ln_body,
        grid=(spb, nb),
        in_specs=[
            pl.BlockSpec((TB, D), tok),
            pl.BlockSpec((TB, D), lambda j, i: (j, 0)),
            pl.BlockSpec((2, D), lambda j, i: (0, 0)),
            pl.BlockSpec((TB, 1), tok),
            pl.BlockSpec((1, D), lambda j, i: (0, 0)),
            pl.BlockSpec((1, D), lambda j, i: (0, 0)),
        ],
        out_specs=pl.BlockSpec((TB, D), tok),
        out_shape=jax.ShapeDtypeStruct((ntok, D), jnp.float32),
    )


def kernel(input_ids, token_type_ids, word_emb, pos_emb, tt_emb, gamma, beta):
    b, seq = input_ids.shape
    ntok = b * seq
    ids_flat = input_ids.reshape(ntok).astype(jnp.int32)
    ttf = token_type_ids.reshape(ntok, 1).astype(jnp.float32)
    wsum = _make_sc_gather(ntok)(ids_flat, word_emb)
    out = _make_tc_ln(ntok)(wsum, pos_emb, tt_emb, ttf,
                            gamma.reshape(1, D), beta.reshape(1, D))
    return out.reshape(b, seq, D)
